# Initial kernel scaffold; baseline (speedup 1.0000x reference)
import jax
import jax.numpy as jnp
from jax.experimental import pallas as pl


def kernel(x, weights, bias):
    raise NotImplementedError("write your pallas kernel here")


# speedup vs baseline: 1.1032x; 1.1032x over previous
import functools

import jax
import jax.numpy as jnp
from jax import lax
from jax.experimental import pallas as pl
from jax.experimental.pallas import tpu as pltpu
from jax.experimental.pallas import tpu_sc as plsc

_UNITS = 26
_LOWER = -17.0
_UPPER = 8.0
_BW = 0.05
_NUM_BUCKETS = int((_UPPER - _LOWER) / _BW) + 1
_RESIDUE = _LOWER - _BW
_BATCH = 4096

_KPAD = 512
_NW = 32
_ELEMS = _BATCH * _UNITS
_EPW = _ELEMS // _NW
_VECS = _EPW // 16


def _table_kernel(w_ref, b_ref, t1_ref, t2_ref):
    w = jnp.maximum(w_ref[...], jnp.float32(0.0))
    r = lax.broadcasted_iota(jnp.int32, (_KPAD, _KPAD), 0)
    c = lax.broadcasted_iota(jnp.int32, (_KPAD, _KPAD), 1)
    tri = jnp.where(r < c, jnp.float32(_BW), jnp.float32(0.0))
    t1_ref[...] = (
        jnp.dot(w, tri, preferred_element_type=jnp.float32,
                precision=lax.Precision.HIGHEST)
        + (jnp.float32(_RESIDUE) + b_ref[...])
    )
    t2_ref[...] = w


def _build_tables(weights, bias):
    wp = jnp.pad(weights, ((0, 0), (0, _KPAD - _NUM_BUCKETS)))
    t1, t2 = pl.pallas_call(
        _table_kernel,
        out_shape=[
            jax.ShapeDtypeStruct((_UNITS, _KPAD), jnp.float32),
            jax.ShapeDtypeStruct((_UNITS, _KPAD), jnp.float32),
        ],
    )(wp, bias.reshape(_UNITS, 1))
    return t1.reshape(-1), t2.reshape(-1)


def _sc_body(x_hbm, t1_hbm, t2_hbm, out_hbm, x_v, out_v, t1_v, t2_v):
    wid = lax.axis_index("s") * 2 + lax.axis_index("c")
    base = wid * _EPW
    pltpu.sync_copy(x_hbm.at[pl.ds(base, _EPW)], x_v)
    pltpu.sync_copy(t1_hbm, t1_v)
    pltpu.sync_copy(t2_hbm, t2_v)

    lane = lax.iota(jnp.int32, 16)

    def body(i, _):
        off = i * 16
        xv = x_v[pl.ds(off, 16)]
        xc = jnp.clip(xv, jnp.float32(_LOWER + 1e-9), jnp.float32(_UPPER - 1e-9))
        t = (xc - jnp.float32(_LOWER) + jnp.float32(_BW)) / jnp.float32(_BW)
        idx = jnp.clip(t.astype(jnp.int32), 0, _NUM_BUCKETS - 1)
        delta = (
            xc - jnp.float32(_LOWER) + jnp.float32(_BW)
            - idx.astype(jnp.float32) * jnp.float32(_BW)
        )
        u = jnp.remainder(base + off + lane, jnp.int32(_UNITS))
        fidx = u * _KPAD + idx
        g1 = plsc.load_gather(t1_v, [fidx])
        g2 = plsc.load_gather(t2_v, [fidx])
        z = g1 + delta * g2
        out_v[pl.ds(off, 16)] = jnp.float32(1.0) / (jnp.float32(1.0) + jnp.exp(-z))
        return _

    lax.fori_loop(0, _VECS, body, None)
    pltpu.sync_copy(out_v, out_hbm.at[pl.ds(base, _EPW)])


def kernel(x, weights, bias):
    t1, t2 = _build_tables(weights, bias)
    mesh = plsc.VectorSubcoreMesh(core_axis_name="c", subcore_axis_name="s")
    run = functools.partial(
        pl.kernel,
        mesh=mesh,
        out_type=jax.ShapeDtypeStruct((_ELEMS,), jnp.float32),
        scratch_types=[
            pltpu.VMEM((_EPW,), jnp.float32),
            pltpu.VMEM((_EPW,), jnp.float32),
            pltpu.VMEM((_UNITS * _KPAD,), jnp.float32),
            pltpu.VMEM((_UNITS * _KPAD,), jnp.float32),
        ],
        compiler_params=pltpu.CompilerParams(needs_layout_passes=False),
    )(_sc_body)
    out = run(x.reshape(-1), t1, t2)
    return out.reshape(_BATCH, _UNITS)
